# baseline (speedup 1.0000x reference)
import functools

import jax
import jax.numpy as jnp
from jax import lax
from jax.experimental import pallas as pl
from jax.experimental.pallas import tpu as pltpu
from jax.experimental.pallas import tpu_sc as plsc

_NUM_STEPS = 1000
_B, _L, _D = 64, 4096, 128
_TL = 4096
_NB = 2
_LANES = 128


def _make_packed_table():
    betas = jnp.linspace(0.0001, 0.02, _NUM_STEPS)
    alphas_bar = jnp.cumprod(1.0 - betas)
    sa = jnp.sqrt(alphas_bar)
    sb = jnp.sqrt(1.0 - alphas_bar)
    return jnp.stack([sa, sb] + [jnp.zeros_like(sa)] * (_LANES - 2), axis=1)


def _sc_gather_body(table_hbm, t_hbm, out_hbm, idx_v, rows_v, sem):
    wid = lax.axis_index("s") * 2 + lax.axis_index("c")

    @pl.when(wid == 0)
    def _():
        pltpu.sync_copy(t_hbm, idx_v)
        pltpu.async_copy(table_hbm.at[idx_v], rows_v, sem).wait()
        pltpu.sync_copy(rows_v, out_hbm)


def _sc_gather(table, t):
    mesh = plsc.VectorSubcoreMesh(core_axis_name="c", subcore_axis_name="s")
    return pl.kernel(
        _sc_gather_body,
        mesh=mesh,
        out_type=jax.ShapeDtypeStruct((_B, _LANES), jnp.float32),
        scratch_types=[
            pltpu.VMEM((_B,), jnp.int32),
            pltpu.VMEM((_B, _LANES), jnp.float32),
            pltpu.SemaphoreType.DMA,
        ],
    )(table, t)


def _blend_body(sab_ref, x_ref, n_ref, o_ref):
    b = pl.program_id(0)
    for i in range(_NB):
        sa = sab_ref[b * _NB + i, 0]
        sb = sab_ref[b * _NB + i, 1]
        o_ref[i] = sa * x_ref[i] + sb * n_ref[i]


def kernel(x, noise, t):
    t = t.astype(jnp.int32)
    sab = _sc_gather(_make_packed_table(), t)
    grid = (_B // _NB,)
    smem = pl.BlockSpec(memory_space=pltpu.SMEM)
    big = pl.BlockSpec((_NB, _TL, _D), lambda b: (b, 0, 0))
    return pl.pallas_call(
        _blend_body,
        grid=grid,
        in_specs=[smem, big, big],
        out_specs=big,
        out_shape=jax.ShapeDtypeStruct((_B, _L, _D), jnp.float32),
    )(sab, x, noise)


# speedup vs baseline: 1.0085x; 1.0085x over previous
import jax
import jax.numpy as jnp
from jax import lax
from jax.experimental import pallas as pl
from jax.experimental.pallas import tpu as pltpu
from jax.experimental.pallas import tpu_sc as plsc

_NUM_STEPS = 1000
_B, _L, _D = 64, 4096, 128
_TL = 4096
_NB = 2
_LANES = 128
_B_SPLIT = 48


def _make_packed_table():
    betas = jnp.linspace(0.0001, 0.02, _NUM_STEPS)
    alphas_bar = jnp.cumprod(1.0 - betas)
    sa = jnp.sqrt(alphas_bar)
    sb = jnp.sqrt(1.0 - alphas_bar)
    return jnp.stack([sa, sb] + [jnp.zeros_like(sa)] * (_LANES - 2), axis=1)


def _sc_gather_body(table_hbm, t_hbm, out_hbm, idx_v, rows_v, sem):
    wid = lax.axis_index("s") * 2 + lax.axis_index("c")

    @pl.when(wid == 0)
    def _():
        pltpu.sync_copy(t_hbm, idx_v)
        pltpu.async_copy(table_hbm.at[idx_v], rows_v, sem).wait()
        pltpu.sync_copy(rows_v, out_hbm)


def _sc_gather(table, t):
    mesh = plsc.VectorSubcoreMesh(core_axis_name="c", subcore_axis_name="s")
    return pl.kernel(
        _sc_gather_body,
        mesh=mesh,
        out_type=jax.ShapeDtypeStruct((_B, _LANES), jnp.float32),
        scratch_types=[
            pltpu.VMEM((_B,), jnp.int32),
            pltpu.VMEM((_B, _LANES), jnp.float32),
            pltpu.SemaphoreType.DMA,
        ],
    )(table, t)


def _blend_lo_body(t_ref, sa_tab_ref, sb_tab_ref, x_ref, n_ref, o_ref):
    b = pl.program_id(0)
    for i in range(_NB):
        tb = t_ref[b * _NB + i]
        sa = sa_tab_ref[tb]
        sb = sb_tab_ref[tb]
        o_ref[i] = sa * x_ref[i] + sb * n_ref[i]


def _blend_hi_body(sab_ref, x_ref, n_ref, prev_ref, o_ref):
    b = pl.program_id(0)
    for i in range(_NB):
        r = _B_SPLIT + b * _NB + i
        sa = sab_ref[r, 0]
        sb = sab_ref[r, 1]
        o_ref[i] = sa * x_ref[i] + sb * n_ref[i]


def kernel(x, noise, t):
    t = t.astype(jnp.int32)
    table = _make_packed_table()
    sa_tab = table[:, 0]
    sb_tab = table[:, 1]
    sab = _sc_gather(table, t)

    smem = pl.BlockSpec(memory_space=pltpu.SMEM)
    big = pl.BlockSpec((_NB, _TL, _D), lambda b: (b, 0, 0))
    out_lo = pl.pallas_call(
        _blend_lo_body,
        grid=(_B_SPLIT // _NB,),
        in_specs=[smem, smem, smem, big, big],
        out_specs=big,
        out_shape=jax.ShapeDtypeStruct((_B, _L, _D), jnp.float32),
    )(t, sa_tab, sb_tab, x, noise)

    hi_blocks = _B_SPLIT // _NB
    big_hi = pl.BlockSpec((_NB, _TL, _D), lambda b: (hi_blocks + b, 0, 0))
    anyspec = pl.BlockSpec(memory_space=pltpu.MemorySpace.HBM)
    return pl.pallas_call(
        _blend_hi_body,
        grid=((_B - _B_SPLIT) // _NB,),
        in_specs=[smem, big_hi, big_hi, anyspec],
        out_specs=big_hi,
        out_shape=jax.ShapeDtypeStruct((_B, _L, _D), jnp.float32),
        input_output_aliases={3: 0},
    )(sab, x, noise, out_lo)


# speedup vs baseline: 1.1629x; 1.1530x over previous
import jax
import jax.numpy as jnp
from jax import lax
from jax.experimental import pallas as pl
from jax.experimental.pallas import tpu as pltpu
from jax.experimental.pallas import tpu_sc as plsc

_NUM_STEPS = 1000
_B, _L, _D = 64, 4096, 128
_TL = 4096
_NB = 2
_LANES = 128
_B_SPLIT = 48


def _make_packed_table():
    betas = jnp.linspace(0.0001, 0.02, _NUM_STEPS)
    alphas_bar = jnp.cumprod(1.0 - betas)
    sa = jnp.sqrt(alphas_bar)
    sb = jnp.sqrt(1.0 - alphas_bar)
    return jnp.stack([sa, sb] + [jnp.zeros_like(sa)] * (_LANES - 2), axis=1)


def _sc_gather_body(table_hbm, t_hbm, out_hbm, idx_v, rows_v, sem):
    wid = lax.axis_index("s") * 2 + lax.axis_index("c")

    @pl.when(wid == 0)
    def _():
        pltpu.sync_copy(t_hbm, idx_v)
        pltpu.async_copy(table_hbm.at[idx_v], rows_v, sem).wait()
        pltpu.sync_copy(rows_v, out_hbm)


def _sc_gather(table, t):
    mesh = plsc.VectorSubcoreMesh(core_axis_name="c", subcore_axis_name="s")
    return pl.kernel(
        _sc_gather_body,
        mesh=mesh,
        out_type=jax.ShapeDtypeStruct((_B, _LANES), jnp.float32),
        scratch_types=[
            pltpu.VMEM((_B,), jnp.int32),
            pltpu.VMEM((_B, _LANES), jnp.float32),
            pltpu.SemaphoreType.DMA,
        ],
    )(table, t)


def _blend_lo_body(t_ref, sa_tab_ref, sb_tab_ref, x_ref, n_ref, o_ref):
    b = pl.program_id(0)
    for i in range(_NB):
        tb = t_ref[b * _NB + i]
        sa = sa_tab_ref[tb]
        sb = sb_tab_ref[tb]
        o_ref[i] = sa * x_ref[i] + sb * n_ref[i]


def _blend_hi_body(t_ref, sa_tab_ref, sb_tab_ref, x_ref, n_ref, prev_ref, o_ref):
    b = pl.program_id(0)
    for i in range(_NB):
        r = _B_SPLIT + b * _NB + i
        tb = t_ref[r]
        sa = sa_tab_ref[tb]
        sb = sb_tab_ref[tb]
        o_ref[i] = sa * x_ref[i] + sb * n_ref[i]


def kernel(x, noise, t):
    t = t.astype(jnp.int32)
    table = _make_packed_table()
    sa_tab = table[:, 0]
    sb_tab = table[:, 1]

    smem = pl.BlockSpec(memory_space=pltpu.SMEM)
    big = pl.BlockSpec((_NB, _TL, _D), lambda b: (b, 0, 0))
    out_lo = pl.pallas_call(
        _blend_lo_body,
        grid=(_B_SPLIT // _NB,),
        in_specs=[smem, smem, smem, big, big],
        out_specs=big,
        out_shape=jax.ShapeDtypeStruct((_B, _L, _D), jnp.float32),
    )(t, sa_tab, sb_tab, x, noise)

    hi_blocks = _B_SPLIT // _NB
    big_hi = pl.BlockSpec((_NB, _TL, _D), lambda b: (hi_blocks + b, 0, 0))
    anyspec = pl.BlockSpec(memory_space=pltpu.MemorySpace.HBM)
    return pl.pallas_call(
        _blend_hi_body,
        grid=((_B - _B_SPLIT) // _NB,),
        in_specs=[smem, smem, smem, big_hi, big_hi, anyspec],
        out_specs=big_hi,
        out_shape=jax.ShapeDtypeStruct((_B, _L, _D), jnp.float32),
        input_output_aliases={5: 0},
    )(t, sa_tab, sb_tab, x, noise, out_lo)
